# baseline (speedup 1.0000x reference)
import functools

import jax
import jax.numpy as jnp
from jax import lax
from jax.experimental import pallas as pl
from jax.experimental.pallas import tpu as pltpu
from jax.experimental.pallas import tpu_sc as plsc

N = 10000
NPAD = 10240
E = 320000
G = 64
NC = 2
NS = 16
ROWS_PER_TILE = NPAD // NS
K = 128
EPT = E // (NC * NS)
FULL = EPT // K
TAIL = EPT - FULL * K
BN = 2048
NB = NPAD // BN


def _sc_pass(table, edge_index, zeros, width, gather=True):
    if width > 16:
        R = 2
    elif gather:
        R = 4
    else:
        R = 6
    I = R + 1
    PERIOD = R * I
    nsteps = -(-(FULL + R) // PERIOD)

    mesh = plsc.VectorSubcoreMesh(core_axis_name="c", subcore_axis_name="s")

    @functools.partial(
        pl.kernel,
        out_type=jax.ShapeDtypeStruct((NC * NPAD, width), jnp.float32),
        mesh=mesh,
        scratch_types=[
            pltpu.VMEM((I, K), jnp.int32),
            pltpu.VMEM((I, K), jnp.int32),
            pltpu.VMEM((R, K, width), jnp.float32),
            pltpu.VMEM((2, TAIL), jnp.int32),
            pltpu.VMEM((TAIL, width), jnp.float32),
            pltpu.VMEM_SHARED((NPAD, width), jnp.float32),
            pltpu.SemaphoreType.DMA((I,)),
            pltpu.SemaphoreType.DMA((R,)),
            pltpu.SemaphoreType.DMA((R,)),
        ],
        compiler_params=pltpu.CompilerParams(use_tc_tiling_on_sc=False),
    )
    def k(table_h, ei_h, zero_h, out_h, src_v, dst_v, rows_v, tidx_v, trows_v,
          agg_sh, sem_i, sem_g, sem_s):
        c = lax.axis_index("c")
        s = lax.axis_index("s")
        r0 = s * ROWS_PER_TILE
        for j in range(ROWS_PER_TILE // K):
            pltpu.sync_copy(zero_h, agg_sh.at[pl.ds(r0 + j * K, K)])
        if not gather:
            pltpu.sync_copy(table_h.at[pl.ds(0, K)], rows_v.at[0])
        plsc.subcore_barrier()

        ebase = (c * NS + s) * EPT

        def idx_copies(g, ib):
            out = [pltpu.make_async_copy(
                ei_h.at[1, pl.ds(ebase + g * K, K)], dst_v.at[ib],
                sem_i.at[ib])]
            if gather:
                out.append(pltpu.make_async_copy(
                    ei_h.at[0, pl.ds(ebase + g * K, K)], src_v.at[ib],
                    sem_i.at[ib]))
            return out

        def gather_desc(b, ib):
            return pltpu.make_async_copy(
                table_h.at[src_v.at[ib]], rows_v.at[b], sem_g.at[b])

        def scatter_desc(b, ib):
            rb = b if gather else 0
            return pltpu.make_async_copy(
                rows_v.at[rb], agg_sh.at[dst_v.at[ib]], sem_s.at[b])

        for d in idx_copies(0, 0):
            d.start()

        def step(t, carry):
            for u in range(PERIOD):
                g = t * PERIOD + u
                b = u % R
                ib = u % I

                @pl.when((g >= R) & (g <= FULL + R - 1))
                def _c():
                    scatter_desc(b, (u - R) % I).wait()

                @pl.when(g < FULL)
                def _a():
                    for d in idx_copies(g, ib):
                        d.wait()
                    if gather:
                        gather_desc(b, ib).start()
                    else:
                        scatter_desc(b, ib).start(add=True)

                if gather:
                    @pl.when((g >= 1) & (g <= FULL))
                    def _b():
                        gather_desc((u - 1) % R, (u - 1) % I).wait()
                        scatter_desc((u - 1) % R, (u - 1) % I).start(add=True)

                @pl.when(g + 1 < FULL)
                def _d():
                    for d in idx_copies(g + 1, (u + 1) % I):
                        d.start()
            return carry

        lax.fori_loop(0, nsteps, step, 0)

        toff = ebase + FULL * K
        pltpu.sync_copy(ei_h.at[1, pl.ds(toff, TAIL)], tidx_v.at[1])
        if gather:
            pltpu.sync_copy(ei_h.at[0, pl.ds(toff, TAIL)], tidx_v.at[0])
            pltpu.sync_copy(table_h.at[tidx_v.at[0]], trows_v)
            pltpu.sync_copy(trows_v, agg_sh.at[tidx_v.at[1]], add=True)
        else:
            pltpu.sync_copy(rows_v.at[0].at[pl.ds(0, TAIL)],
                            agg_sh.at[tidx_v.at[1]], add=True)

        plsc.subcore_barrier()
        pltpu.sync_copy(agg_sh.at[pl.ds(r0, ROWS_PER_TILE)],
                        out_h.at[pl.ds(c * NPAD + r0, ROWS_PER_TILE)])

    return k(table, edge_index, zeros)


def _t1a(xp, W1):
    def body(x_ref, w_ref, out_ref):
        out_ref[...] = jnp.dot(x_ref[...], w_ref[...],
                               preferred_element_type=jnp.float32)

    return pl.pallas_call(
        body,
        grid=(NB,),
        in_specs=[
            pl.BlockSpec((BN, 128), lambda i: (i, 0)),
            pl.BlockSpec((128, 128), lambda i: (0, 0)),
        ],
        out_specs=pl.BlockSpec((BN, 128), lambda i: (i, 0)),
        out_shape=jax.ShapeDtypeStruct((NPAD, 128), jnp.float32),
    )(xp, W1)


def _t1b(xw, cnt):
    def body(xw_ref, c0_ref, c1_ref, h_ref, dinv_ref):
        deg = c0_ref[...][:, :1] + c1_ref[...][:, :1] + 1.0
        dinv = lax.rsqrt(deg)
        dinv_ref[...] = dinv
        h_ref[...] = xw_ref[...] * dinv

    return pl.pallas_call(
        body,
        grid=(NB,),
        in_specs=[
            pl.BlockSpec((BN, 128), lambda i: (i, 0)),
            pl.BlockSpec((BN, 16), lambda i: (i, 0)),
            pl.BlockSpec((BN, 16), lambda i: (NB + i, 0)),
        ],
        out_specs=[
            pl.BlockSpec((BN, 128), lambda i: (i, 0)),
            pl.BlockSpec((BN, 1), lambda i: (i, 0)),
        ],
        out_shape=[
            jax.ShapeDtypeStruct((NPAD, 128), jnp.float32),
            jax.ShapeDtypeStruct((NPAD, 1), jnp.float32),
        ],
    )(xw, cnt, cnt)


def _mid(agg, hp, dinv, b, W):
    outw = W.shape[1]

    def body(a0_ref, a1_ref, hp_ref, dinv_ref, b_ref, w_ref, out_ref):
        s = a0_ref[...] + a1_ref[...] + hp_ref[...]
        dinv = dinv_ref[...]
        z = jnp.maximum(s * dinv + b_ref[...], 0.0)
        out_ref[...] = jnp.dot(z, w_ref[...],
                               preferred_element_type=jnp.float32) * dinv

    return pl.pallas_call(
        body,
        grid=(NB,),
        in_specs=[
            pl.BlockSpec((BN, 128), lambda i: (i, 0)),
            pl.BlockSpec((BN, 128), lambda i: (NB + i, 0)),
            pl.BlockSpec((BN, 128), lambda i: (i, 0)),
            pl.BlockSpec((BN, 1), lambda i: (i, 0)),
            pl.BlockSpec((1, 128), lambda i: (0, 0)),
            pl.BlockSpec((128, outw), lambda i: (0, 0)),
        ],
        out_specs=pl.BlockSpec((BN, outw), lambda i: (i, 0)),
        out_shape=jax.ShapeDtypeStruct((NPAD, outw), jnp.float32),
    )(agg, agg, hp, dinv, b, W)


def _t4(agg, hp, dinv, b3p, batr):
    def body(g0_ref, g1_ref, hp_ref, dinv_ref, b_ref, bat_ref, out_ref):
        i = pl.program_id(0)

        @pl.when(i == 0)
        def _init():
            out_ref[...] = jnp.zeros_like(out_ref)

        p = ((g0_ref[...] + g1_ref[...] + hp_ref[...]) * dinv_ref[...]
             + b_ref[...])
        col = lax.broadcasted_iota(jnp.int32, (BN, 16), 1)
        p_aug = jnp.where(col == 15, 1.0, p)
        gids = lax.broadcasted_iota(jnp.int32, (G, BN), 0)
        mask = (bat_ref[...] == gids).astype(jnp.float32)
        out_ref[...] += jnp.dot(mask, p_aug,
                                preferred_element_type=jnp.float32)

        @pl.when(i == NB - 1)
        def _final():
            sums = out_ref[...]
            cnt = jnp.maximum(sums[:, 15:16], 1.0)
            m = sums / cnt
            ccol = lax.broadcasted_iota(jnp.int32, (G, 16), 1)
            logits = jnp.where(ccol < 10, m, -1e30)
            zz = logits - jnp.max(logits, axis=1, keepdims=True)
            ez = jnp.exp(zz)
            out_ref[...] = ez / jnp.sum(ez, axis=1, keepdims=True)

    return pl.pallas_call(
        body,
        grid=(NB,),
        in_specs=[
            pl.BlockSpec((BN, 16), lambda i: (i, 0)),
            pl.BlockSpec((BN, 16), lambda i: (NB + i, 0)),
            pl.BlockSpec((BN, 16), lambda i: (i, 0)),
            pl.BlockSpec((BN, 1), lambda i: (i, 0)),
            pl.BlockSpec((1, 16), lambda i: (0, 0)),
            pl.BlockSpec((1, BN), lambda i: (0, i)),
        ],
        out_specs=pl.BlockSpec((G, 16), lambda i: (0, 0)),
        out_shape=jax.ShapeDtypeStruct((G, 16), jnp.float32),
    )(agg, agg, hp, dinv, b3p, batr)


def kernel(x, edge_index, batch, W1, b1, W2, b2, W3, b3):
    xp = jnp.zeros((NPAD, 128), jnp.float32).at[:N].set(x)
    batr = jnp.full((NPAD,), G, jnp.int32).at[:N].set(batch).reshape(1, NPAD)
    W3p = jnp.zeros((128, 16), jnp.float32).at[:, :10].set(W3)
    b1r = b1.reshape(1, 128)
    b2r = b2.reshape(1, 128)
    b3p = jnp.zeros((1, 16), jnp.float32).at[0, :10].set(b3)
    ones16 = jnp.ones((NPAD, 16), jnp.float32)
    z16 = jnp.zeros((K, 16), jnp.float32)
    z128 = jnp.zeros((K, 128), jnp.float32)

    cnt = _sc_pass(ones16, edge_index, z16, 16, gather=False)
    xw = _t1a(xp, W1)
    h1, dinv = _t1b(xw, cnt)
    agg1 = _sc_pass(h1, edge_index, z128, 128)
    h2 = _mid(agg1, h1, dinv, b1r, W2)
    agg2 = _sc_pass(h2, edge_index, z128, 128)
    h3 = _mid(agg2, h2, dinv, b2r, W3p)
    agg3 = _sc_pass(h3, edge_index, z16, 16)
    out = _t4(agg3, h3, dinv, b3p, batr)
    return out[:, :10]


# speedup vs baseline: 1.0375x; 1.0375x over previous
import functools

import jax
import jax.numpy as jnp
from jax import lax
from jax.experimental import pallas as pl
from jax.experimental.pallas import tpu as pltpu
from jax.experimental.pallas import tpu_sc as plsc

N = 10000
NPAD = 10240
E = 320000
G = 64
NC = 2
NS = 16
ROWS_PER_TILE = NPAD // NS
K = 128
EPT = E // (NC * NS)
FULL = EPT // K
TAIL = EPT - FULL * K
BN = 2048
NB = NPAD // BN


def _sc_pass(table, edge_index, zeros, width, gather=True):
    R = 2
    I = R + 1
    PERIOD = R * I
    nsteps = -(-(FULL + R) // PERIOD)

    mesh = plsc.VectorSubcoreMesh(core_axis_name="c", subcore_axis_name="s")

    @functools.partial(
        pl.kernel,
        out_type=jax.ShapeDtypeStruct((NC * NPAD, width), jnp.float32),
        mesh=mesh,
        scratch_types=[
            pltpu.VMEM((I, K), jnp.int32),
            pltpu.VMEM((I, K), jnp.int32),
            pltpu.VMEM((R, K, width), jnp.float32),
            pltpu.VMEM((2, TAIL), jnp.int32),
            pltpu.VMEM((TAIL, width), jnp.float32),
            pltpu.VMEM_SHARED((NPAD, width), jnp.float32),
            pltpu.SemaphoreType.DMA((I,)),
            pltpu.SemaphoreType.DMA((R,)),
            pltpu.SemaphoreType.DMA((R,)),
        ],
        compiler_params=pltpu.CompilerParams(use_tc_tiling_on_sc=False),
    )
    def k(table_h, ei_h, zero_h, out_h, src_v, dst_v, rows_v, tidx_v, trows_v,
          agg_sh, sem_i, sem_g, sem_s):
        c = lax.axis_index("c")
        s = lax.axis_index("s")
        r0 = s * ROWS_PER_TILE
        pltpu.sync_copy(zero_h.at[pl.ds(r0, ROWS_PER_TILE)],
                        agg_sh.at[pl.ds(r0, ROWS_PER_TILE)])
        if not gather:
            pltpu.sync_copy(table_h.at[pl.ds(0, K)], rows_v.at[0])
        plsc.subcore_barrier()

        ebase = (c * NS + s) * EPT

        def idx_copies(g, ib):
            out = [pltpu.make_async_copy(
                ei_h.at[1, pl.ds(ebase + g * K, K)], dst_v.at[ib],
                sem_i.at[ib])]
            if gather:
                out.append(pltpu.make_async_copy(
                    ei_h.at[0, pl.ds(ebase + g * K, K)], src_v.at[ib],
                    sem_i.at[ib]))
            return out

        def gather_desc(b, ib):
            return pltpu.make_async_copy(
                table_h.at[src_v.at[ib]], rows_v.at[b], sem_g.at[b])

        def scatter_desc(b, ib):
            rb = b if gather else 0
            return pltpu.make_async_copy(
                rows_v.at[rb], agg_sh.at[dst_v.at[ib]], sem_s.at[b])

        for d in idx_copies(0, 0):
            d.start()

        def step(t, carry):
            for u in range(PERIOD):
                g = t * PERIOD + u
                b = u % R
                ib = u % I

                @pl.when((g >= R) & (g <= FULL + R - 1))
                def _c():
                    scatter_desc(b, (u - R) % I).wait()

                @pl.when(g < FULL)
                def _a():
                    for d in idx_copies(g, ib):
                        d.wait()
                    if gather:
                        gather_desc(b, ib).start()
                    else:
                        scatter_desc(b, ib).start(add=True)

                if gather:
                    @pl.when((g >= 1) & (g <= FULL))
                    def _b():
                        gather_desc((u - 1) % R, (u - 1) % I).wait()
                        scatter_desc((u - 1) % R, (u - 1) % I).start(add=True)

                @pl.when(g + 1 < FULL)
                def _d():
                    for d in idx_copies(g + 1, (u + 1) % I):
                        d.start()
            return carry

        lax.fori_loop(0, nsteps, step, 0)

        toff = ebase + FULL * K
        pltpu.sync_copy(ei_h.at[1, pl.ds(toff, TAIL)], tidx_v.at[1])
        if gather:
            pltpu.sync_copy(ei_h.at[0, pl.ds(toff, TAIL)], tidx_v.at[0])
            pltpu.sync_copy(table_h.at[tidx_v.at[0]], trows_v)
            pltpu.sync_copy(trows_v, agg_sh.at[tidx_v.at[1]], add=True)
        else:
            pltpu.sync_copy(rows_v.at[0].at[pl.ds(0, TAIL)],
                            agg_sh.at[tidx_v.at[1]], add=True)

        plsc.subcore_barrier()
        pltpu.sync_copy(agg_sh.at[pl.ds(r0, ROWS_PER_TILE)],
                        out_h.at[pl.ds(c * NPAD + r0, ROWS_PER_TILE)])

    return k(table, edge_index, zeros)


def _t1a(xp, W1):
    def body(x_ref, w_ref, out_ref):
        out_ref[...] = jnp.dot(x_ref[...], w_ref[...],
                               preferred_element_type=jnp.float32)

    return pl.pallas_call(
        body,
        grid=(NB,),
        in_specs=[
            pl.BlockSpec((BN, 128), lambda i: (i, 0)),
            pl.BlockSpec((128, 128), lambda i: (0, 0)),
        ],
        out_specs=pl.BlockSpec((BN, 128), lambda i: (i, 0)),
        out_shape=jax.ShapeDtypeStruct((NPAD, 128), jnp.float32),
    )(xp, W1)


def _t1b(xw, c0, c1):
    def body(xw_ref, c0_ref, c1_ref, h_ref, dinv_ref):
        deg = c0_ref[...] + c1_ref[...] + 1.0
        dinv = lax.rsqrt(deg)
        dinv_ref[...] = dinv
        h_ref[...] = xw_ref[...] * dinv

    return pl.pallas_call(
        body,
        grid=(NB,),
        in_specs=[
            pl.BlockSpec((BN, 128), lambda i: (i, 0)),
            pl.BlockSpec((BN, 1), lambda i: (i, 0)),
            pl.BlockSpec((BN, 1), lambda i: (i, 0)),
        ],
        out_specs=[
            pl.BlockSpec((BN, 128), lambda i: (i, 0)),
            pl.BlockSpec((BN, 1), lambda i: (i, 0)),
        ],
        out_shape=[
            jax.ShapeDtypeStruct((NPAD, 128), jnp.float32),
            jax.ShapeDtypeStruct((NPAD, 1), jnp.float32),
        ],
    )(xw, c0, c1)


def _mid(agg, hp, dinv, b, W):
    outw = W.shape[1]

    def body(a0_ref, a1_ref, hp_ref, dinv_ref, b_ref, w_ref, out_ref):
        s = a0_ref[...] + a1_ref[...] + hp_ref[...]
        dinv = dinv_ref[...]
        z = jnp.maximum(s * dinv + b_ref[...], 0.0)
        out_ref[...] = jnp.dot(z, w_ref[...],
                               preferred_element_type=jnp.float32) * dinv

    return pl.pallas_call(
        body,
        grid=(NB,),
        in_specs=[
            pl.BlockSpec((BN, 128), lambda i: (i, 0)),
            pl.BlockSpec((BN, 128), lambda i: (NB + i, 0)),
            pl.BlockSpec((BN, 128), lambda i: (i, 0)),
            pl.BlockSpec((BN, 1), lambda i: (i, 0)),
            pl.BlockSpec((1, 128), lambda i: (0, 0)),
            pl.BlockSpec((128, outw), lambda i: (0, 0)),
        ],
        out_specs=pl.BlockSpec((BN, outw), lambda i: (i, 0)),
        out_shape=jax.ShapeDtypeStruct((NPAD, outw), jnp.float32),
    )(agg, agg, hp, dinv, b, W)


def _t4(agg, hp, dinv, b3p, batr):
    def body(g0_ref, g1_ref, hp_ref, dinv_ref, b_ref, bat_ref, out_ref):
        i = pl.program_id(0)

        @pl.when(i == 0)
        def _init():
            out_ref[...] = jnp.zeros_like(out_ref)

        p = ((g0_ref[...] + g1_ref[...] + hp_ref[...]) * dinv_ref[...]
             + b_ref[...])
        col = lax.broadcasted_iota(jnp.int32, (BN, 16), 1)
        p_aug = jnp.where(col == 15, 1.0, p)
        gids = lax.broadcasted_iota(jnp.int32, (G, BN), 0)
        mask = (bat_ref[...] == gids).astype(jnp.float32)
        out_ref[...] += jnp.dot(mask, p_aug,
                                preferred_element_type=jnp.float32)

        @pl.when(i == NB - 1)
        def _final():
            sums = out_ref[...]
            cnt = jnp.maximum(sums[:, 15:16], 1.0)
            m = sums / cnt
            ccol = lax.broadcasted_iota(jnp.int32, (G, 16), 1)
            logits = jnp.where(ccol < 10, m, -1e30)
            zz = logits - jnp.max(logits, axis=1, keepdims=True)
            ez = jnp.exp(zz)
            out_ref[...] = ez / jnp.sum(ez, axis=1, keepdims=True)

    return pl.pallas_call(
        body,
        grid=(NB,),
        in_specs=[
            pl.BlockSpec((BN, 16), lambda i: (i, 0)),
            pl.BlockSpec((BN, 16), lambda i: (NB + i, 0)),
            pl.BlockSpec((BN, 16), lambda i: (i, 0)),
            pl.BlockSpec((BN, 1), lambda i: (i, 0)),
            pl.BlockSpec((1, 16), lambda i: (0, 0)),
            pl.BlockSpec((1, BN), lambda i: (0, i)),
        ],
        out_specs=pl.BlockSpec((G, 16), lambda i: (0, 0)),
        out_shape=jax.ShapeDtypeStruct((G, 16), jnp.float32),
    )(agg, agg, hp, dinv, b3p, batr)


def kernel(x, edge_index, batch, W1, b1, W2, b2, W3, b3):
    xp = jnp.zeros((NPAD, 128), jnp.float32).at[:N].set(x)
    batr = jnp.full((NPAD,), G, jnp.int32).at[:N].set(batch).reshape(1, NPAD)
    W3p = jnp.zeros((128, 16), jnp.float32).at[:, :10].set(W3)
    b1r = b1.reshape(1, 128)
    b2r = b2.reshape(1, 128)
    b3p = jnp.zeros((1, 16), jnp.float32).at[0, :10].set(b3)
    ones16 = jnp.ones((NPAD, 16), jnp.float32)
    z16 = jnp.zeros((NPAD, 16), jnp.float32)
    z128 = jnp.zeros((NPAD, 128), jnp.float32)

    cnt = _sc_pass(ones16, edge_index, z16, 16, gather=False)
    xw = _t1a(xp, W1)
    h1, dinv = _t1b(xw, cnt[:NPAD, :1], cnt[NPAD:, :1])
    agg1 = _sc_pass(h1, edge_index, z128, 128)
    h2 = _mid(agg1, h1, dinv, b1r, W2)
    agg2 = _sc_pass(h2, edge_index, z128, 128)
    h3 = _mid(agg2, h2, dinv, b2r, W3p)
    agg3 = _sc_pass(h3, edge_index, z16, 16)
    out = _t4(agg3, h3, dinv, b3p, batr)
    return out[:, :10]


# speedup vs baseline: 1.1713x; 1.1289x over previous
import functools

import jax
import jax.numpy as jnp
from jax import lax
from jax.experimental import pallas as pl
from jax.experimental.pallas import tpu as pltpu
from jax.experimental.pallas import tpu_sc as plsc

N = 10000
NPAD = 10240
E = 320000
G = 64
NC = 2
NS = 16
ROWS_PER_TILE = NPAD // NS
K = 128
EPT = E // (NC * NS)
FULL = EPT // K
TAIL = EPT - FULL * K
BN = 2048
NB = NPAD // BN


CH = E // K
CBASE = CH // (NC * NS)
XTRA = CH - CBASE * (NC * NS)
CMAX = CBASE + 1


def _sc_pass(table, ei3, zeros, width, gather=True):
    if not gather:
        R = 8
    elif width > 16:
        R = 2
    else:
        R = 6
    PERIOD = 6 if (gather and width > 16) else R

    nsteps = -(-(CMAX + R) // PERIOD)
    mesh = plsc.VectorSubcoreMesh(core_axis_name="c", subcore_axis_name="s")

    scratch = [pltpu.VMEM((CMAX, K), jnp.int32)]
    if gather and width <= 16:
        scratch.append(pltpu.VMEM((CMAX, K), jnp.int32))
    if gather and width > 16:
        scratch.append(pltpu.VMEM((3, K), jnp.int32))
    scratch += [
        pltpu.VMEM((R if gather else 1, K, width), jnp.float32),
        pltpu.VMEM_SHARED((NPAD, width), jnp.float32),
        pltpu.SemaphoreType.DMA((3,)),
        pltpu.SemaphoreType.DMA((R,)),
        pltpu.SemaphoreType.DMA((R,)),
    ]

    @functools.partial(
        pl.kernel,
        out_type=jax.ShapeDtypeStruct((NC * NPAD, width), jnp.float32),
        mesh=mesh,
        scratch_types=scratch,
        compiler_params=pltpu.CompilerParams(use_tc_tiling_on_sc=False),
    )
    def k(table_h, ei_h, zero_h, out_h, dst_all, *rest):
        if gather and width <= 16:
            src_all = rest[0]
            rest = rest[1:]
        if gather and width > 16:
            src_v = rest[0]
            rest = rest[1:]
        rows_v, agg_sh, sem_i, sem_g, sem_s = rest

        c = lax.axis_index("c")
        s = lax.axis_index("s")
        w = c * NS + s
        r0 = s * ROWS_PER_TILE
        cb = CBASE * w + jnp.minimum(w, XTRA)
        nck = CBASE + (w < XTRA).astype(jnp.int32)

        pltpu.sync_copy(ei_h.at[1, pl.ds(cb, CBASE)],
                        dst_all.at[pl.ds(0, CBASE)])
        if gather and width <= 16:
            pltpu.sync_copy(ei_h.at[0, pl.ds(cb, CBASE)],
                            src_all.at[pl.ds(0, CBASE)])

        @pl.when(w < XTRA)
        def _extra():
            pltpu.sync_copy(ei_h.at[1, pl.ds(cb + CBASE, 1)],
                            dst_all.at[pl.ds(CBASE, 1)])
            if gather and width <= 16:
                pltpu.sync_copy(ei_h.at[0, pl.ds(cb + CBASE, 1)],
                                src_all.at[pl.ds(CBASE, 1)])

        pltpu.sync_copy(zero_h.at[pl.ds(r0, ROWS_PER_TILE)],
                        agg_sh.at[pl.ds(r0, ROWS_PER_TILE)])
        if not gather:
            pltpu.sync_copy(table_h.at[pl.ds(0, K)], rows_v.at[0])
        plsc.subcore_barrier()

        if not gather:
            def sc_desc(g, b):
                return pltpu.make_async_copy(
                    rows_v.at[0], agg_sh.at[dst_all.at[g]], sem_s.at[b])

            def step(t, carry):
                for u in range(PERIOD):
                    g = t * PERIOD + u

                    @pl.when((g >= R) & (g <= nck + R - 1))
                    def _w():
                        sc_desc(g - R, u).wait()

                    @pl.when(g < nck)
                    def _s():
                        sc_desc(g, u).start(add=True)
                return carry

        elif width <= 16:
            def ga_desc(g, b):
                return pltpu.make_async_copy(
                    table_h.at[src_all.at[g]], rows_v.at[b], sem_g.at[b])

            def sc_desc(g, b):
                return pltpu.make_async_copy(
                    rows_v.at[b], agg_sh.at[dst_all.at[g]], sem_s.at[b])

            def step(t, carry):
                for u in range(PERIOD):
                    g = t * PERIOD + u
                    um1 = (u - 1) % R

                    @pl.when((g >= R) & (g <= nck + R - 1))
                    def _c():
                        sc_desc(g - R, u).wait()

                    @pl.when(g < nck)
                    def _a():
                        ga_desc(g, u).start()

                    @pl.when((g >= 1) & (g <= nck))
                    def _b():
                        ga_desc(g - 1, um1).wait()
                        sc_desc(g - 1, um1).start(add=True)
                return carry

        else:
            def src_desc(g, ib):
                return pltpu.make_async_copy(
                    ei_h.at[0, cb + g], src_v.at[ib], sem_i.at[ib])

            def ga_desc(b, ib):
                return pltpu.make_async_copy(
                    table_h.at[src_v.at[ib]], rows_v.at[b], sem_g.at[b])

            def sc_desc(g, b):
                return pltpu.make_async_copy(
                    rows_v.at[b], agg_sh.at[dst_all.at[g]], sem_s.at[b])

            src_desc(0, 0).start()

            def step(t, carry):
                for u in range(PERIOD):
                    g = t * PERIOD + u
                    b = u % 2
                    ib = u % 3

                    @pl.when((g >= 2) & (g <= nck + 1))
                    def _c():
                        sc_desc(g - 2, b).wait()

                    @pl.when(g < nck)
                    def _a():
                        src_desc(g, ib).wait()
                        ga_desc(b, ib).start()

                    @pl.when((g >= 1) & (g <= nck))
                    def _b():
                        ga_desc(1 - b, (u - 1) % 3).wait()
                        sc_desc(g - 1, 1 - b).start(add=True)

                    @pl.when(g + 1 < nck)
                    def _d():
                        src_desc(g + 1, (u + 1) % 3).start()
                return carry

        lax.fori_loop(0, nsteps, step, 0)
        plsc.subcore_barrier()
        pltpu.sync_copy(agg_sh.at[pl.ds(r0, ROWS_PER_TILE)],
                        out_h.at[pl.ds(c * NPAD + r0, ROWS_PER_TILE)])

    return k(table, ei3, zeros)


def _t1a(xp, W1):
    def body(x_ref, w_ref, out_ref):
        out_ref[...] = jnp.dot(x_ref[...], w_ref[...],
                               preferred_element_type=jnp.float32)

    return pl.pallas_call(
        body,
        grid=(NB,),
        in_specs=[
            pl.BlockSpec((BN, 128), lambda i: (i, 0)),
            pl.BlockSpec((128, 128), lambda i: (0, 0)),
        ],
        out_specs=pl.BlockSpec((BN, 128), lambda i: (i, 0)),
        out_shape=jax.ShapeDtypeStruct((NPAD, 128), jnp.float32),
    )(xp, W1)


def _t1b(xw, c0, c1):
    def body(xw_ref, c0_ref, c1_ref, h_ref, dinv_ref):
        deg = c0_ref[...] + c1_ref[...] + 1.0
        dinv = lax.rsqrt(deg)
        dinv_ref[...] = dinv
        h_ref[...] = xw_ref[...] * dinv

    return pl.pallas_call(
        body,
        grid=(NB,),
        in_specs=[
            pl.BlockSpec((BN, 128), lambda i: (i, 0)),
            pl.BlockSpec((BN, 1), lambda i: (i, 0)),
            pl.BlockSpec((BN, 1), lambda i: (i, 0)),
        ],
        out_specs=[
            pl.BlockSpec((BN, 128), lambda i: (i, 0)),
            pl.BlockSpec((BN, 1), lambda i: (i, 0)),
        ],
        out_shape=[
            jax.ShapeDtypeStruct((NPAD, 128), jnp.float32),
            jax.ShapeDtypeStruct((NPAD, 1), jnp.float32),
        ],
    )(xw, c0, c1)


def _mid(agg, hp, dinv, b, W):
    outw = W.shape[1]

    def body(a0_ref, a1_ref, hp_ref, dinv_ref, b_ref, w_ref, out_ref):
        s = a0_ref[...] + a1_ref[...] + hp_ref[...]
        dinv = dinv_ref[...]
        z = jnp.maximum(s * dinv + b_ref[...], 0.0)
        out_ref[...] = jnp.dot(z, w_ref[...],
                               preferred_element_type=jnp.float32) * dinv

    return pl.pallas_call(
        body,
        grid=(NB,),
        in_specs=[
            pl.BlockSpec((BN, 128), lambda i: (i, 0)),
            pl.BlockSpec((BN, 128), lambda i: (NB + i, 0)),
            pl.BlockSpec((BN, 128), lambda i: (i, 0)),
            pl.BlockSpec((BN, 1), lambda i: (i, 0)),
            pl.BlockSpec((1, 128), lambda i: (0, 0)),
            pl.BlockSpec((128, outw), lambda i: (0, 0)),
        ],
        out_specs=pl.BlockSpec((BN, outw), lambda i: (i, 0)),
        out_shape=jax.ShapeDtypeStruct((NPAD, outw), jnp.float32),
    )(agg, agg, hp, dinv, b, W)


def _t4(agg, hp, dinv, b3p, batr):
    def body(g0_ref, g1_ref, hp_ref, dinv_ref, b_ref, bat_ref, out_ref):
        i = pl.program_id(0)

        @pl.when(i == 0)
        def _init():
            out_ref[...] = jnp.zeros_like(out_ref)

        p = ((g0_ref[...] + g1_ref[...] + hp_ref[...]) * dinv_ref[...]
             + b_ref[...])
        col = lax.broadcasted_iota(jnp.int32, (BN, 16), 1)
        p_aug = jnp.where(col == 15, 1.0, p)
        gids = lax.broadcasted_iota(jnp.int32, (G, BN), 0)
        mask = (bat_ref[...] == gids).astype(jnp.float32)
        out_ref[...] += jnp.dot(mask, p_aug,
                                preferred_element_type=jnp.float32)

        @pl.when(i == NB - 1)
        def _final():
            sums = out_ref[...]
            cnt = jnp.maximum(sums[:, 15:16], 1.0)
            m = sums / cnt
            ccol = lax.broadcasted_iota(jnp.int32, (G, 16), 1)
            logits = jnp.where(ccol < 10, m, -1e30)
            zz = logits - jnp.max(logits, axis=1, keepdims=True)
            ez = jnp.exp(zz)
            out_ref[...] = ez / jnp.sum(ez, axis=1, keepdims=True)

    return pl.pallas_call(
        body,
        grid=(NB,),
        in_specs=[
            pl.BlockSpec((BN, 16), lambda i: (i, 0)),
            pl.BlockSpec((BN, 16), lambda i: (NB + i, 0)),
            pl.BlockSpec((BN, 16), lambda i: (i, 0)),
            pl.BlockSpec((BN, 1), lambda i: (i, 0)),
            pl.BlockSpec((1, 16), lambda i: (0, 0)),
            pl.BlockSpec((1, BN), lambda i: (0, i)),
        ],
        out_specs=pl.BlockSpec((G, 16), lambda i: (0, 0)),
        out_shape=jax.ShapeDtypeStruct((G, 16), jnp.float32),
    )(agg, agg, hp, dinv, b3p, batr)


def kernel(x, edge_index, batch, W1, b1, W2, b2, W3, b3):
    xp = jnp.zeros((NPAD, 128), jnp.float32).at[:N].set(x)
    batr = jnp.full((NPAD,), G, jnp.int32).at[:N].set(batch).reshape(1, NPAD)
    W3p = jnp.zeros((128, 16), jnp.float32).at[:, :10].set(W3)
    b1r = b1.reshape(1, 128)
    b2r = b2.reshape(1, 128)
    b3p = jnp.zeros((1, 16), jnp.float32).at[0, :10].set(b3)
    ones16 = jnp.ones((NPAD, 16), jnp.float32)
    z16 = jnp.zeros((NPAD, 16), jnp.float32)
    z128 = jnp.zeros((NPAD, 128), jnp.float32)

    ei3 = edge_index.reshape(2, CH, K)

    cnt = _sc_pass(ones16, ei3, z16, 16, gather=False)
    xw = _t1a(xp, W1)
    h1, dinv = _t1b(xw, cnt[:NPAD, :1], cnt[NPAD:, :1])
    agg1 = _sc_pass(h1, ei3, z128, 128)
    h2 = _mid(agg1, h1, dinv, b1r, W2)
    agg2 = _sc_pass(h2, ei3, z128, 128)
    h3 = _mid(agg2, h2, dinv, b2r, W3p)
    agg3 = _sc_pass(h3, ei3, z16, 16)
    out = _t4(agg3, h3, dinv, b3p, batr)
    return out[:, :10]
